# baseline (speedup 1.0000x reference)
import functools

import jax
import jax.numpy as jnp
from jax import lax
from jax.experimental import pallas as pl
from jax.experimental.pallas import tpu as pltpu
from jax.experimental.pallas import tpu_sc as plsc

NUM_EMB = 1000000
DIM = 64
B_TOTAL = 16384 * 50

NC = 2
NS = 16
NW = NC * NS

CHUNK = 128
B_PER_W = B_TOTAL // NW
NCHUNKS = B_PER_W // CHUNK


NBUF = 8
AHEAD = NBUF - 1


def _gather_kernel(table_hbm, idx_hbm, out_hbm, idx_v, bufs, gsems, ssems):
    wid = lax.axis_index("s") * NC + lax.axis_index("c")
    base = wid * B_PER_W

    pltpu.sync_copy(idx_hbm.at[wid], idx_v)

    def gather(c, b):
        return pltpu.make_async_copy(table_hbm.at[idx_v.at[c]], bufs[b], gsems[b])

    def store(c, b):
        return pltpu.make_async_copy(
            bufs[b], out_hbm.at[pl.ds(base + c * CHUNK, CHUNK)], ssems[b]
        )

    for c in range(AHEAD):
        gather(c, c).start()

    def step(i, carry):
        g = i * NBUF
        for b in range(NBUF):
            c = g + b
            gather(c, b).wait()
            store(c, b).start()

            pb = (b - 1) % NBUF

            @pl.when(c >= 1)
            def _():
                store(c - 1, pb).wait()

            @pl.when(c + AHEAD < NCHUNKS)
            def _():
                gather(c + AHEAD, pb).start()
        return carry

    lax.fori_loop(0, NCHUNKS // NBUF, step, 0)
    store(NCHUNKS - 1, (NCHUNKS - 1) % NBUF).wait()


@functools.partial(
    pl.kernel,
    mesh=plsc.VectorSubcoreMesh(core_axis_name="c", subcore_axis_name="s"),
    out_type=jax.ShapeDtypeStruct((B_TOTAL, DIM), jnp.float32),
    scratch_types=[
        pltpu.VMEM((NCHUNKS, CHUNK), jnp.int32),
        [pltpu.VMEM((CHUNK, DIM), jnp.float32)] * NBUF,
        [pltpu.SemaphoreType.DMA] * NBUF,
        [pltpu.SemaphoreType.DMA] * NBUF,
    ],
    compiler_params=pltpu.CompilerParams(use_tc_tiling_on_sc=False),
)
def _lookup(table_hbm, idx_hbm, out_hbm, idx_v, bufs, gsems, ssems):
    _gather_kernel(table_hbm, idx_hbm, out_hbm, idx_v, bufs, gsems, ssems)


def kernel(x, table):
    batch, hist = x.shape
    idx = jnp.reshape(x.astype(jnp.int32), (NW, NCHUNKS, CHUNK))
    out = _lookup(table, idx)
    return jnp.reshape(out, (batch, hist, DIM))


# speedup vs baseline: 1.2671x; 1.2671x over previous
import functools

import jax
import jax.numpy as jnp
from jax import lax
from jax.experimental import pallas as pl
from jax.experimental.pallas import tpu as pltpu
from jax.experimental.pallas import tpu_sc as plsc

NUM_EMB = 1000000
DIM = 64
BATCH = 16384
HIST = 50
B_TOTAL = BATCH * HIST

NC = 2
NS = 16
NW = NC * NS

CHUNK = 128
B_PER_W = B_TOTAL // NW
NCHUNKS = B_PER_W // CHUNK
IBLKS = BATCH // CHUNK

NBUF = 4
AHEAD = NBUF - 1
PITCH = 129


def _gather_kernel(table_hbm, idx_hbm, out_hbm, idx_v, bufs, tbufs, gsems, ssems):
    wid = lax.axis_index("s") * NC + lax.axis_index("c")
    base = wid * NCHUNKS

    pltpu.sync_copy(idx_hbm.at[wid], idx_v)

    lanes = lax.iota(jnp.int32, 16)
    khv = [(jnp.int32(g * 16) + lanes) >> 3 for g in range(4)]
    klv = [(jnp.int32(g * 16) + lanes) & 7 for g in range(4)]

    def gather(c, b):
        return pltpu.make_async_copy(table_hbm.at[idx_v.at[c]], bufs[b], gsems[b])

    def store(c, b):
        f = base + c
        j = f // IBLKS
        ib = f % IBLKS
        return pltpu.make_async_copy(
            tbufs[b].at[:, :, pl.ds(0, CHUNK)], out_hbm.at[j, :, ib], ssems[b]
        )

    for c in range(AHEAD):
        gather(c, c).start()

    def step(i, carry):
        g0 = i * NBUF
        for b in range(NBUF):
            c = g0 + b
            gather(c, b).wait()

            rows = bufs[b]
            tb = tbufs[b]

            def trow(r, carry2):
                for u in range(4):
                    il = r * 4 + u
                    ilv = jnp.full((16,), 0, jnp.int32) + il
                    for g in range(4):
                        vals = rows[il, pl.ds(g * 16, 16)]
                        plsc.store_scatter(tb, [khv[g], klv[g], ilv], vals)
                return carry2

            lax.fori_loop(0, CHUNK // 4, trow, 0)

            store(c, b).start()

            pb = (b - 1) % NBUF

            @pl.when(c >= 1)
            def _():
                store(c - 1, pb).wait()

            @pl.when(c + AHEAD < NCHUNKS)
            def _():
                gather(c + AHEAD, pb).start()
        return carry

    lax.fori_loop(0, NCHUNKS // NBUF, step, 0)
    store(NCHUNKS - 1, (NCHUNKS - 1) % NBUF).wait()


@functools.partial(
    pl.kernel,
    mesh=plsc.VectorSubcoreMesh(core_axis_name="c", subcore_axis_name="s"),
    out_type=jax.ShapeDtypeStruct((HIST, DIM // 8, IBLKS, 8, CHUNK), jnp.float32),
    scratch_types=[
        pltpu.VMEM((NCHUNKS, CHUNK), jnp.int32),
        [pltpu.VMEM((CHUNK, DIM), jnp.float32)] * NBUF,
        [pltpu.VMEM((DIM // 8, 8, PITCH), jnp.float32)] * NBUF,
        [pltpu.SemaphoreType.DMA] * NBUF,
        [pltpu.SemaphoreType.DMA] * NBUF,
    ],
    compiler_params=pltpu.CompilerParams(
        use_tc_tiling_on_sc=False, needs_layout_passes=False
    ),
)
def _lookup(table_hbm, idx_hbm, out_hbm, idx_v, bufs, tbufs, gsems, ssems):
    _gather_kernel(table_hbm, idx_hbm, out_hbm, idx_v, bufs, tbufs, gsems, ssems)


def kernel(x, table):
    batch, hist = x.shape
    idx = jnp.reshape(jnp.transpose(x).astype(jnp.int32), (NW, NCHUNKS, CHUNK))
    r5 = _lookup(table, idx)
    return jnp.reshape(jnp.transpose(r5, (2, 4, 0, 1, 3)), (batch, hist, DIM))


# speedup vs baseline: 1.2725x; 1.0043x over previous
import functools

import jax
import jax.numpy as jnp
from jax import lax
from jax.experimental import pallas as pl
from jax.experimental.pallas import tpu as pltpu
from jax.experimental.pallas import tpu_sc as plsc

NUM_EMB = 1000000
DIM = 64
BATCH = 16384
HIST = 50
B_TOTAL = BATCH * HIST

NC = 2
NS = 16
NW = NC * NS

CHUNK = 128
B_PER_W = B_TOTAL // NW
NCHUNKS = B_PER_W // CHUNK
IBLKS = BATCH // CHUNK

NBUF = 4
AHEAD = NBUF - 1
PITCH = 129


def _gather_kernel(table_hbm, idx_hbm, out_hbm, idx_v, bufs, tbufs, gsems, ssems):
    wid = lax.axis_index("s") * NC + lax.axis_index("c")
    base = wid * NCHUNKS

    pltpu.sync_copy(idx_hbm.at[wid], idx_v)

    lanes = lax.iota(jnp.int32, 16)
    khv = [(jnp.int32(g * 16) + lanes) >> 3 for g in range(4)]
    klv = [(jnp.int32(g * 16) + lanes) & 7 for g in range(4)]

    def gather(c, b):
        return pltpu.make_async_copy(table_hbm.at[idx_v.at[c]], bufs[b], gsems[b])

    def store(c, b):
        f = base + c
        j = f // IBLKS
        ib = f % IBLKS
        return pltpu.make_async_copy(
            tbufs[b].at[:, :, pl.ds(0, CHUNK)], out_hbm.at[j, :, ib], ssems[b]
        )

    for c in range(AHEAD):
        gather(c, c).start()

    def step(i, carry):
        g0 = i * NBUF
        for b in range(NBUF):
            c = g0 + b
            gather(c, b).wait()

            rows = bufs[b]
            tb = tbufs[b]

            def trow(r, carry2):
                for u in range(8):
                    il = r * 8 + u
                    ilv = jnp.full((16,), 0, jnp.int32) + il
                    for g in range(4):
                        vals = rows[il, pl.ds(g * 16, 16)]
                        plsc.store_scatter(tb, [khv[g], klv[g], ilv], vals)
                return carry2

            lax.fori_loop(0, CHUNK // 8, trow, 0)

            store(c, b).start()

            pb = (b - 1) % NBUF

            @pl.when(c >= 1)
            def _():
                store(c - 1, pb).wait()

            @pl.when(c + AHEAD < NCHUNKS)
            def _():
                gather(c + AHEAD, pb).start()
        return carry

    lax.fori_loop(0, NCHUNKS // NBUF, step, 0)
    store(NCHUNKS - 1, (NCHUNKS - 1) % NBUF).wait()


@functools.partial(
    pl.kernel,
    mesh=plsc.VectorSubcoreMesh(core_axis_name="c", subcore_axis_name="s"),
    out_type=jax.ShapeDtypeStruct((HIST, DIM // 8, IBLKS, 8, CHUNK), jnp.float32),
    scratch_types=[
        pltpu.VMEM((NCHUNKS, CHUNK), jnp.int32),
        [pltpu.VMEM((CHUNK, DIM), jnp.float32)] * NBUF,
        [pltpu.VMEM((DIM // 8, 8, PITCH), jnp.float32)] * NBUF,
        [pltpu.SemaphoreType.DMA] * NBUF,
        [pltpu.SemaphoreType.DMA] * NBUF,
    ],
    compiler_params=pltpu.CompilerParams(
        use_tc_tiling_on_sc=False, needs_layout_passes=False
    ),
)
def _lookup(table_hbm, idx_hbm, out_hbm, idx_v, bufs, tbufs, gsems, ssems):
    _gather_kernel(table_hbm, idx_hbm, out_hbm, idx_v, bufs, tbufs, gsems, ssems)


def kernel(x, table):
    batch, hist = x.shape
    idx = jnp.reshape(jnp.transpose(x).astype(jnp.int32), (NW, NCHUNKS, CHUNK))
    r5 = _lookup(table, idx)
    return jnp.reshape(jnp.transpose(r5, (2, 4, 0, 1, 3)), (batch, hist, DIM))
